# baseline (speedup 1.0000x reference)
import functools

import jax
import jax.numpy as jnp
from jax import lax
from jax.experimental import pallas as pl
from jax.experimental.pallas import tpu as pltpu
from jax.experimental.pallas import tpu_sc as plsc

B = 16384
D = 64
DW = D // 2
NF = 5
OUT = 128

_NC = 2
_NS = 16
_NW = _NC * _NS
_BPW = B // _NW
_CHUNK = 128
_NCHUNK = _BPW // _CHUNK
_NIT = _NCHUNK * NF
_NBUF = 8


def _sc_gather_body(samp_hbm, m0, m1, m2, m3, m4, t0, t1, t2, t3, t4,
                    oa_hbm, ob_hbm, samp_v, idx_v, rows_v,
                    sem_m, sem_g, sem_w):
    wid = lax.axis_index("s") * _NC + lax.axis_index("c")
    base = wid * _BPW
    maps = (m0, m1, m2, m3, m4)
    tabs = (t0, t1, t2, t3, t4)
    pltpu.sync_copy(samp_hbm.at[pl.ds(wid * _NCHUNK, _NCHUNK)], samp_v)
    mdesc = []
    for i in range(_NIT):
        c, f = divmod(i, NF)
        mdesc.append(
            pltpu.async_copy(maps[f].at[samp_v.at[c]], idx_v.at[i], sem_m))

    def _write(j):
        c, f = divmod(j, NF)
        rsl = pl.ds(base + c * _CHUNK, _CHUNK)
        src = rows_v.at[j % _NBUF]
        if f < NF - 1:
            return [pltpu.async_copy(src, oa_hbm.at[rsl, pl.ds(f * DW, DW)],
                                     sem_w)]
        return [pltpu.async_copy(src, ob_hbm.at[rsl, pl.ds(k * DW, DW)], sem_w)
                for k in range(4)]

    gdesc = [None] * _NIT
    wdesc = [None] * _NIT
    for i in range(_NIT):
        c, f = divmod(i, NF)
        if i >= _NBUF:
            for wd in wdesc[i - _NBUF]:
                wd.wait()
        mdesc[i].wait()
        gdesc[i] = pltpu.async_copy(tabs[f].at[idx_v.at[i]],
                                    rows_v.at[i % _NBUF], sem_g)
        if i >= 1:
            gdesc[i - 1].wait()
            wdesc[i - 1] = _write(i - 1)
    gdesc[_NIT - 1].wait()
    wdesc[_NIT - 1] = _write(_NIT - 1)
    for j in range(_NIT - _NBUF, _NIT):
        for wd in wdesc[j]:
            wd.wait()


_SC_MESH = plsc.VectorSubcoreMesh(core_axis_name="c", subcore_axis_name="s")

_sc_gather = functools.partial(
    pl.kernel,
    out_type=[jax.ShapeDtypeStruct((B, OUT), jnp.float32)] * 2,
    mesh=_SC_MESH,
    scratch_types=[
        pltpu.VMEM((_NCHUNK, _CHUNK), jnp.int32),
        pltpu.VMEM((_NIT, _CHUNK), jnp.int32),
        pltpu.VMEM((_NBUF, _CHUNK, DW), jnp.float32),
        pltpu.SemaphoreType.DMA,
        pltpu.SemaphoreType.DMA,
        pltpu.SemaphoreType.DMA,
    ],
    compiler_params=pltpu.CompilerParams(use_tc_tiling_on_sc=False),
)(_sc_gather_body)


def _mlp_body(xa_ref, xb_ref, w_ref, b_ref, o_ref):
    mask = jnp.int32(-65536)

    def _planes(x):
        xi = lax.bitcast_convert_type(x, jnp.int32)
        even = lax.bitcast_convert_type(xi << 16, jnp.float32)
        odd = lax.bitcast_convert_type(xi & mask, jnp.float32)
        return even, odd

    ea, oa = _planes(xa_ref[...])
    eb, ob = _planes(xb_ref[...])
    x = jnp.concatenate([ea, oa, eb, ob], axis=-1)
    h = jnp.dot(x, w_ref[...],
                preferred_element_type=jnp.float32) + b_ref[...]
    o_ref[...] = h * (1.0 / (1.0 + jnp.exp(-h)))


def _mlp(xa, xb, w_cat, b2d):
    bm = 2048
    return pl.pallas_call(
        _mlp_body,
        grid=(B // bm,),
        in_specs=[
            pl.BlockSpec((bm, OUT), lambda i: (i, 0)),
            pl.BlockSpec((bm, OUT), lambda i: (i, 0)),
            pl.BlockSpec((4 * OUT, OUT), lambda i: (0, 0)),
            pl.BlockSpec((1, OUT), lambda i: (0, 0)),
        ],
        out_specs=pl.BlockSpec((bm, OUT), lambda i: (i, 0)),
        out_shape=jax.ShapeDtypeStruct((B, OUT), jnp.float32),
    )(xa, xb, w_cat, b2d)


def _packed_view(table):
    tb = table.astype(jnp.bfloat16)
    n = table.shape[0]
    return lax.bitcast_convert_type(tb.reshape(n, DW, 2), jnp.float32)


def kernel(sample, map_cat0, map_cat1, map_cat2, map_cat3,
           emb_cat0, emb_cat1, emb_cat2, emb_cat3,
           map_text, text_table, W, b):
    samp2d = sample.astype(jnp.int32).reshape(_NW * _NCHUNK, _CHUNK)
    xa, xb = _sc_gather(
        samp2d,
        map_cat0.astype(jnp.int32), map_cat1.astype(jnp.int32),
        map_cat2.astype(jnp.int32), map_cat3.astype(jnp.int32),
        map_text.astype(jnp.int32),
        _packed_view(emb_cat0), _packed_view(emb_cat1),
        _packed_view(emb_cat2), _packed_view(emb_cat3),
        _packed_view(text_table),
    )
    w4 = W[:4 * D].reshape(4, DW, 2, OUT)
    wea = w4[:, :, 0, :].reshape(4 * DW, OUT)
    woa = w4[:, :, 1, :].reshape(4 * DW, OUT)
    wt = W[4 * D:].reshape(DW, 2, OUT)
    zpad = jnp.zeros((3 * DW, OUT), W.dtype)
    web = jnp.concatenate([wt[:, 0, :], zpad], axis=0)
    wob = jnp.concatenate([wt[:, 1, :], zpad], axis=0)
    w_cat = jnp.concatenate([wea, woa, web, wob], axis=0)
    return _mlp(xa, xb, w_cat, b.reshape(1, OUT))


# speedup vs baseline: 2.3429x; 2.3429x over previous
import functools

import jax
import jax.numpy as jnp
from jax import lax
from jax.experimental import pallas as pl
from jax.experimental.pallas import tpu as pltpu
from jax.experimental.pallas import tpu_sc as plsc

B = 16384
D = 64
DW = D // 2
NF = 5
OUT = 128

_NC = 2
_NS = 16
_NW = _NC * _NS
_BPW = B // _NW
_CHUNK = 128
_NCHUNK = _BPW // _CHUNK
_NIT = _NCHUNK * NF
_NBUF = 8


def _sc_gather_body(samp_hbm, m0, m1, m2, m3, m4, t0, t1, t2, t3, t4,
                    oa_hbm, ob_hbm, samp_v, idx_v, rows_v,
                    sem_m, sem_g, sem_w):
    wid = lax.axis_index("s") * _NC + lax.axis_index("c")
    base = wid * _BPW
    maps = (m0, m1, m2, m3, m4)
    tabs = (t0, t1, t2, t3, t4)
    pltpu.sync_copy(samp_hbm.at[pl.ds(wid * _NCHUNK, _NCHUNK)], samp_v)
    mdesc = []
    for i in range(_NIT):
        c, f = divmod(i, NF)
        mdesc.append(
            pltpu.async_copy(maps[f].at[samp_v.at[c]], idx_v.at[i], sem_m))

    def _write(j):
        c, f = divmod(j, NF)
        rsl = pl.ds(base + c * _CHUNK, _CHUNK)
        src = rows_v.at[j % _NBUF]
        if f < NF - 1:
            return [pltpu.async_copy(src, oa_hbm.at[rsl, pl.ds(f * D, D)],
                                     sem_w)]
        return [pltpu.async_copy(src, ob_hbm.at[rsl, pl.ds(k * D, D)], sem_w)
                for k in range(4)]

    gdesc = [None] * _NIT
    wdesc = [None] * _NIT
    for i in range(_NIT):
        c, f = divmod(i, NF)
        if i >= _NBUF:
            for wd in wdesc[i - _NBUF]:
                wd.wait()
        mdesc[i].wait()
        gdesc[i] = pltpu.async_copy(tabs[f].at[idx_v.at[i]],
                                    rows_v.at[i % _NBUF], sem_g)
        if i >= 1:
            gdesc[i - 1].wait()
            wdesc[i - 1] = _write(i - 1)
    gdesc[_NIT - 1].wait()
    wdesc[_NIT - 1] = _write(_NIT - 1)
    for j in range(_NIT - _NBUF, _NIT):
        for wd in wdesc[j]:
            wd.wait()


_SC_MESH = plsc.VectorSubcoreMesh(core_axis_name="c", subcore_axis_name="s")

_sc_gather = functools.partial(
    pl.kernel,
    out_type=[jax.ShapeDtypeStruct((B, 4 * D), jnp.bfloat16)] * 2,
    mesh=_SC_MESH,
    scratch_types=[
        pltpu.VMEM((_NCHUNK, _CHUNK), jnp.int32),
        pltpu.VMEM((_NIT, _CHUNK), jnp.int32),
        pltpu.VMEM((_NBUF, _CHUNK, D), jnp.bfloat16),
        pltpu.SemaphoreType.DMA,
        pltpu.SemaphoreType.DMA,
        pltpu.SemaphoreType.DMA,
    ],
    compiler_params=pltpu.CompilerParams(use_tc_tiling_on_sc=False),
)(_sc_gather_body)


def _mlp_body(xa_ref, xb_ref, w_ref, b_ref, o_ref):
    x = jnp.concatenate([xa_ref[...], xb_ref[...]],
                        axis=-1).astype(jnp.float32)
    h = jnp.dot(x, w_ref[...],
                preferred_element_type=jnp.float32) + b_ref[...]
    o_ref[...] = h * (1.0 / (1.0 + jnp.exp(-h)))


def _mlp(xa, xb, w_cat, b2d):
    bm = 2048
    return pl.pallas_call(
        _mlp_body,
        grid=(B // bm,),
        in_specs=[
            pl.BlockSpec((bm, 4 * D), lambda i: (i, 0)),
            pl.BlockSpec((bm, 4 * D), lambda i: (i, 0)),
            pl.BlockSpec((8 * D, OUT), lambda i: (0, 0)),
            pl.BlockSpec((1, OUT), lambda i: (0, 0)),
        ],
        out_specs=pl.BlockSpec((bm, OUT), lambda i: (i, 0)),
        out_shape=jax.ShapeDtypeStruct((B, OUT), jnp.float32),
    )(xa, xb, w_cat, b2d)


def _packed_view(table):
    return table.astype(jnp.bfloat16)


def kernel(sample, map_cat0, map_cat1, map_cat2, map_cat3,
           emb_cat0, emb_cat1, emb_cat2, emb_cat3,
           map_text, text_table, W, b):
    samp2d = sample.astype(jnp.int32).reshape(_NW * _NCHUNK, _CHUNK)
    xa, xb = _sc_gather(
        samp2d,
        map_cat0.astype(jnp.int32), map_cat1.astype(jnp.int32),
        map_cat2.astype(jnp.int32), map_cat3.astype(jnp.int32),
        map_text.astype(jnp.int32),
        _packed_view(emb_cat0), _packed_view(emb_cat1),
        _packed_view(emb_cat2), _packed_view(emb_cat3),
        _packed_view(text_table),
    )
    w_cat = jnp.concatenate(
        [W, jnp.zeros((3 * D, OUT), W.dtype)], axis=0)
    return _mlp(xa, xb, w_cat, b.reshape(1, OUT))


# speedup vs baseline: 3.1498x; 1.3444x over previous
import functools

import jax
import jax.numpy as jnp
from jax import lax
from jax.experimental import pallas as pl
from jax.experimental.pallas import tpu as pltpu
from jax.experimental.pallas import tpu_sc as plsc

B = 16384
D = 64
NF = 5
XCOLS = 384
OUT = 128

_NC = 2
_NS = 16
_NW = _NC * _NS
_BPW = B // _NW
_CHUNK = 128
_NCHUNK = _BPW // _CHUNK
_NIT = _NCHUNK * NF
_NBUF = 8


def _sc_gather_body(samp_hbm, m0, m1, m2, m3, m4, t0, t1, t2, t3, t4,
                    oa_hbm, ob_hbm, oc_hbm, samp_v, idx_v, rows_v,
                    sem_m, sem_g, sem_w):
    wid = lax.axis_index("s") * _NC + lax.axis_index("c")
    base = wid * _BPW
    maps = (m0, m1, m2, m3, m4)
    tabs = (t0, t1, t2, t3, t4)
    pltpu.sync_copy(samp_hbm.at[pl.ds(wid * _NCHUNK, _NCHUNK)], samp_v)
    mdesc = []
    for i in range(_NIT):
        c, f = divmod(i, NF)
        mdesc.append(
            pltpu.async_copy(maps[f].at[samp_v.at[c]], idx_v.at[i], sem_m))

    def _write(j):
        c, f = divmod(j, NF)
        out = (oa_hbm, oa_hbm, ob_hbm, ob_hbm, oc_hbm)[f]
        col = (0, D, 0, D, 0)[f]
        rsl = pl.ds(base + c * _CHUNK, _CHUNK)
        w = [pltpu.async_copy(rows_v.at[j % _NBUF], out.at[rsl, pl.ds(col, D)],
                              sem_w)]
        if f == NF - 1:
            w.append(pltpu.async_copy(rows_v.at[j % _NBUF],
                                      oc_hbm.at[rsl, pl.ds(D, D)], sem_w))
        return w

    gdesc = [None] * _NIT
    wdesc = [None] * _NIT
    for i in range(_NIT):
        c, f = divmod(i, NF)
        if i >= _NBUF:
            for wd in wdesc[i - _NBUF]:
                wd.wait()
        mdesc[i].wait()
        gdesc[i] = pltpu.async_copy(tabs[f].at[idx_v.at[i]],
                                    rows_v.at[i % _NBUF], sem_g)
        if i >= 1:
            gdesc[i - 1].wait()
            wdesc[i - 1] = _write(i - 1)
    gdesc[_NIT - 1].wait()
    wdesc[_NIT - 1] = _write(_NIT - 1)
    for j in range(_NIT - _NBUF, _NIT):
        for wd in wdesc[j]:
            wd.wait()


_SC_MESH = plsc.VectorSubcoreMesh(core_axis_name="c", subcore_axis_name="s")

_sc_gather = functools.partial(
    pl.kernel,
    out_type=[jax.ShapeDtypeStruct((B, 2 * D), jnp.float32)] * 3,
    mesh=_SC_MESH,
    scratch_types=[
        pltpu.VMEM((_NCHUNK, _CHUNK), jnp.int32),
        pltpu.VMEM((_NIT, _CHUNK), jnp.int32),
        pltpu.VMEM((_NBUF, _CHUNK, D), jnp.float32),
        pltpu.SemaphoreType.DMA,
        pltpu.SemaphoreType.DMA,
        pltpu.SemaphoreType.DMA,
    ],
    compiler_params=pltpu.CompilerParams(use_tc_tiling_on_sc=False),
)(_sc_gather_body)


def _mlp_body(xa_ref, xb_ref, xc_ref, w_ref, b_ref, o_ref):
    x = jnp.concatenate([xa_ref[...], xb_ref[...], xc_ref[...]], axis=-1)
    h = jnp.dot(x, w_ref[...],
                preferred_element_type=jnp.float32) + b_ref[...]
    o_ref[...] = h * (1.0 / (1.0 + jnp.exp(-h)))


def _mlp(xa, xb, xc, w_pad, b2d):
    bm = 2048
    return pl.pallas_call(
        _mlp_body,
        grid=(B // bm,),
        in_specs=[
            pl.BlockSpec((bm, 2 * D), lambda i: (i, 0)),
            pl.BlockSpec((bm, 2 * D), lambda i: (i, 0)),
            pl.BlockSpec((bm, 2 * D), lambda i: (i, 0)),
            pl.BlockSpec((XCOLS, OUT), lambda i: (0, 0)),
            pl.BlockSpec((1, OUT), lambda i: (0, 0)),
        ],
        out_specs=pl.BlockSpec((bm, OUT), lambda i: (i, 0)),
        out_shape=jax.ShapeDtypeStruct((B, OUT), jnp.float32),
    )(xa, xb, xc, w_pad, b2d)


def kernel(sample, map_cat0, map_cat1, map_cat2, map_cat3,
           emb_cat0, emb_cat1, emb_cat2, emb_cat3,
           map_text, text_table, W, b):
    samp2d = sample.astype(jnp.int32).reshape(_NW * _NCHUNK, _CHUNK)
    xa, xb, xc = _sc_gather(
        samp2d,
        map_cat0.astype(jnp.int32), map_cat1.astype(jnp.int32),
        map_cat2.astype(jnp.int32), map_cat3.astype(jnp.int32),
        map_text.astype(jnp.int32),
        emb_cat0, emb_cat1, emb_cat2, emb_cat3, text_table,
    )
    w_pad = jnp.concatenate([W, jnp.zeros((D, OUT), dtype=W.dtype)], axis=0)
    return _mlp(xa, xb, xc, w_pad, b.reshape(1, OUT))


# speedup vs baseline: 3.2725x; 1.0389x over previous
import functools

import jax
import jax.numpy as jnp
from jax import lax
from jax.experimental import pallas as pl
from jax.experimental.pallas import tpu as pltpu
from jax.experimental.pallas import tpu_sc as plsc

B = 16384
D = 64
NF = 5
OUT = 128

_NC = 2
_NS = 16
_NW = _NC * _NS
_BPW = B // _NW
_CHUNK = 128
_NCHUNK = _BPW // _CHUNK
_NIT = _NCHUNK * NF
_NBUF = 6
_L = 16


def _sc_gather_body(samp_hbm, m0, m1, m2, m3, m4, t0, t1, t2, t3, t4,
                    o0, o1, o2, o3, o4, samp_v, idx_v, rows_v,
                    sem_m, sem_g, sem_w):
    wid = lax.axis_index("s") * _NC + lax.axis_index("c")
    base = wid * _BPW
    maps = (m0, m1, m2, m3, m4)
    tabs = (t0, t1, t2, t3, t4)
    outs = (o0, o1, o2, o3, o4)
    pltpu.sync_copy(samp_hbm.at[pl.ds(wid * _NCHUNK, _NCHUNK)], samp_v)
    mdesc = []
    for i in range(_NIT):
        c, f = divmod(i, NF)
        mdesc.append(
            pltpu.async_copy(maps[f].at[samp_v.at[c]], idx_v.at[i], sem_m))

    def _write(j):
        c, f = divmod(j, NF)
        rsl = pl.ds(base + c * _CHUNK, _CHUNK)
        return pltpu.async_copy(rows_v.at[j % _NBUF], outs[f].at[rsl], sem_w)

    gdesc = [None] * _NIT
    wdesc = [None] * _NIT
    for i in range(_NIT):
        if i >= _NBUF:
            wdesc[i - _NBUF].wait()
        mdesc[i].wait()
        gdesc[i] = pltpu.async_copy(tabs[divmod(i, NF)[1]].at[idx_v.at[i]],
                                    rows_v.at[i % _NBUF], sem_g)
        if i >= 1:
            gdesc[i - 1].wait()
            wdesc[i - 1] = _write(i - 1)
    gdesc[_NIT - 1].wait()
    wdesc[_NIT - 1] = _write(_NIT - 1)
    for j in range(_NIT - _NBUF, _NIT):
        wdesc[j].wait()


_SC_MESH = plsc.VectorSubcoreMesh(core_axis_name="c", subcore_axis_name="s")

_sc_gather = functools.partial(
    pl.kernel,
    out_type=[jax.ShapeDtypeStruct((B, 2 * D), jnp.float32)] * NF,
    mesh=_SC_MESH,
    scratch_types=[
        pltpu.VMEM((_NCHUNK, _CHUNK), jnp.int32),
        pltpu.VMEM((_NIT, _CHUNK), jnp.int32),
        pltpu.VMEM((_NBUF, _CHUNK, 2 * D), jnp.float32),
        pltpu.SemaphoreType.DMA,
        pltpu.SemaphoreType.DMA,
        pltpu.SemaphoreType.DMA,
    ],
    compiler_params=pltpu.CompilerParams(use_tc_tiling_on_sc=False,
                                         needs_layout_passes=False),
)(_sc_gather_body)


def _mlp_body(x0, x1, x2, x3, x4, w_ref, b_ref, o_ref):
    x = jnp.concatenate(
        [x0[...], x1[...], x2[...], x3[...], x4[...]], axis=-1)
    h = jnp.dot(x, w_ref[...],
                preferred_element_type=jnp.float32) + b_ref[...]
    o_ref[...] = h * (1.0 / (1.0 + jnp.exp(-h)))


def _mlp(feats, w2, b2d):
    bm = 2048
    in_specs = [pl.BlockSpec((bm, 2 * D), lambda i: (i, 0))
                for _ in range(NF)]
    in_specs += [
        pl.BlockSpec((NF * 2 * D, OUT), lambda i: (0, 0)),
        pl.BlockSpec((1, OUT), lambda i: (0, 0)),
    ]
    return pl.pallas_call(
        _mlp_body,
        grid=(B // bm,),
        in_specs=in_specs,
        out_specs=pl.BlockSpec((bm, OUT), lambda i: (i, 0)),
        out_shape=jax.ShapeDtypeStruct((B, OUT), jnp.float32),
    )(*feats, w2, b2d)


def _padded(table):
    return jnp.pad(table, ((0, 0), (0, D)))


def kernel(sample, map_cat0, map_cat1, map_cat2, map_cat3,
           emb_cat0, emb_cat1, emb_cat2, emb_cat3,
           map_text, text_table, W, b):
    samp2d = sample.astype(jnp.int32).reshape(_NW * _NCHUNK, _CHUNK)
    feats = _sc_gather(
        samp2d,
        map_cat0.astype(jnp.int32), map_cat1.astype(jnp.int32),
        map_cat2.astype(jnp.int32), map_cat3.astype(jnp.int32),
        map_text.astype(jnp.int32),
        _padded(emb_cat0), _padded(emb_cat1), _padded(emb_cat2),
        _padded(emb_cat3), _padded(text_table),
    )
    zd = jnp.zeros((D, OUT), W.dtype)
    w2 = jnp.concatenate(
        [m for f in range(NF) for m in (W[f * D:(f + 1) * D], zd)], axis=0)
    return _mlp(feats, w2, b.reshape(1, OUT))
